# baseline (speedup 1.0000x reference)
import functools

import jax
import jax.numpy as jnp
import numpy as np
from jax import lax
from jax.experimental import pallas as pl
from jax.experimental.pallas import tpu as pltpu
from jax.experimental.pallas import tpu_sc as plsc

N_NODES = 10000
D_IN = 16
D_OUT = 16
HIDDEN = 256

_NC = 2
_NS = 16
_SCP = pltpu.CompilerParams(use_tc_tiling_on_sc=False)


def _sc_gather(table, idx, base_edge, n_edges, chunk=2000):
    D = D_IN
    nw = _NC * _NS
    per_w = n_edges // nw
    n_ch = per_w // chunk
    ckr = chunk * D // 128
    mesh = plsc.VectorSubcoreMesh(core_axis_name="c", subcore_axis_name="s")

    @functools.partial(
        pl.kernel,
        mesh=mesh,
        out_type=jax.ShapeDtypeStruct((n_edges // chunk, ckr, 128), jnp.float32),
        scratch_types=[
            pltpu.VMEM((chunk,), jnp.int32),
            pltpu.VMEM((chunk, D), jnp.float32),
            pltpu.VMEM((ckr, 128), jnp.float32),
            pltpu.SemaphoreType.DMA,
        ],
        compiler_params=_SCP,
    )
    def k(table_hbm, idx_hbm, out_hbm, idx_v, rows_v, packed_v, sem):
        wid = lax.axis_index("s") * _NC + lax.axis_index("c")
        base = wid * per_w

        def body(i, carry):
            off = base + i * chunk
            pltpu.sync_copy(idx_hbm.at[pl.ds(base_edge + off, chunk)], idx_v)
            pltpu.async_copy(table_hbm.at[idx_v], rows_v, sem).wait()

            def pack(j, c2):
                for l in range(8):
                    packed_v[j, pl.ds(l * D, D)] = rows_v[l * ckr + j, :]
                return c2

            lax.fori_loop(0, ckr, pack, 0)
            pltpu.sync_copy(packed_v, out_hbm.at[off // chunk])
            return carry

        lax.fori_loop(0, n_ch, body, 0)

    return k(table, idx)


def _sc_scatter_add(rows_packed, idx, base_edge, n_out, chunk=2000):
    D = D_OUT
    E = rows_packed.shape[0] * chunk
    per_core = E // _NC
    per_w = per_core // _NS
    n_ch = per_w // chunk
    ckr = chunk * D // 128
    rpt = ((n_out // _NS) // 8) * 8
    rem = n_out - rpt * _NS
    out_rows = 1280
    mesh = plsc.VectorSubcoreMesh(core_axis_name="c", subcore_axis_name="s")

    @functools.partial(
        pl.kernel,
        mesh=mesh,
        out_type=jax.ShapeDtypeStruct((_NC, out_rows, 128), jnp.float32),
        scratch_types=[
            pltpu.VMEM((chunk,), jnp.int32),
            pltpu.VMEM((ckr, 128), jnp.float32),
            pltpu.VMEM((chunk, D), jnp.float32),
            pltpu.VMEM((rpt + rem, D), jnp.float32),
            pltpu.VMEM(((rpt + rem) * D // 128, 128), jnp.float32),
            pltpu.VMEM_SHARED((n_out, D), jnp.float32),
            pltpu.SemaphoreType.DMA,
        ],
        compiler_params=_SCP,
    )
    def k(rp_hbm, idx_hbm, zeros_hbm, out_hbm, idx_v, packed_v, rows_v, bounce,
          bpk, accum, sem):
        c = lax.axis_index("c")
        s = lax.axis_index("s")
        base = c * per_core + s * per_w
        zoff = s * rpt

        pltpu.sync_copy(
            zeros_hbm.at[pl.ds(zoff, rpt)],
            accum.at[pl.ds(zoff, rpt)],
        )

        @pl.when(s == _NS - 1)
        def _zero_tail():
            pltpu.sync_copy(
                zeros_hbm.at[pl.ds(rpt * _NS, rem)],
                accum.at[pl.ds(rpt * _NS, rem)],
            )

        plsc.subcore_barrier()

        def body(i, carry):
            off = base + i * chunk
            pltpu.sync_copy(idx_hbm.at[pl.ds(base_edge + off, chunk)], idx_v)
            pltpu.sync_copy(rp_hbm.at[off // chunk], packed_v)

            def unpack(j, c2):
                for l in range(8):
                    rows_v[l * ckr + j, :] = packed_v[j, pl.ds(l * D, D)]
                return c2

            lax.fori_loop(0, ckr, unpack, 0)
            pltpu.sync_copy(rows_v, accum.at[idx_v], add=True)
            return carry

        lax.fori_loop(0, n_ch, body, 0)
        plsc.subcore_barrier()

        def flush(src_off, nrows, dst_row):
            pltpu.sync_copy(accum.at[pl.ds(src_off, nrows)], bounce.at[pl.ds(0, nrows)])

            def pack(j, c2):
                for l in range(8):
                    bpk[j, pl.ds(l * D, D)] = bounce[j * 8 + l, :]
                return c2

            lax.fori_loop(0, nrows * D // 128, pack, 0)
            pltpu.sync_copy(
                bpk.at[pl.ds(0, nrows * D // 128)],
                out_hbm.at[c, pl.ds(dst_row, nrows * D // 128)],
            )

        flush(zoff, rpt, (zoff * D) // 128)

        @pl.when(s == _NS - 1)
        def _tail():
            flush(rpt * _NS, rem, (rpt * _NS * D) // 128)

    zeros = jnp.zeros((n_out, D), jnp.float32)
    return k(rows_packed, idx, zeros)


def _tc_dense(ell, xp, w1n, w2n, block_base=0, block=2000):
    pk = block * D_IN // 128
    sub = block // 8
    grid = xp.shape[0]
    assert xp.shape == (grid, pk, 128)

    R = np.zeros((8, 128, HIDDEN), np.float32)
    U = np.zeros((8, HIDDEN, 128), np.float32)
    for l in range(8):
        for i in range(D_IN):
            for k in range(D_OUT):
                R[l, 16 * l + i, 16 * i + k] = 1.0
                U[l, 16 * i + k, 16 * l + k] = 1.0
    rcat = jnp.asarray(R.reshape(8 * 128, HIDDEN))
    ucat = jnp.asarray(U.reshape(8 * HIDDEN, 128))

    def body(lsh_ref, x_ref, w1_ref, w2_ref, r_ref, u_ref, o_ref):
        lsh = lsh_ref[...].astype(jnp.float32)
        h = jnp.maximum(
            jnp.dot(lsh, w1_ref[...], preferred_element_type=jnp.float32), 0.0
        )
        wt = jnp.dot(h, w2_ref[...], preferred_element_type=jnp.float32)
        wts = wt * lsh[:, 3:4]
        xpb = x_ref[0]
        o = None
        for l in range(8):
            xr = jnp.dot(
                xpb, r_ref[128 * l : 128 * (l + 1), :],
                preferred_element_type=jnp.float32,
            )
            y = xr * wts[sub * l : sub * (l + 1), :]
            t = jnp.dot(
                y, u_ref[HIDDEN * l : HIDDEN * (l + 1), :],
                preferred_element_type=jnp.float32,
            )
            o = t if o is None else o + t
        o_ref[0] = o

    return pl.pallas_call(
        body,
        grid=(grid,),
        in_specs=[
            pl.BlockSpec((block, 4), lambda i: (i + block_base, 0)),
            pl.BlockSpec((1, pk, 128), lambda i: (i, 0, 0)),
            pl.BlockSpec((4, HIDDEN), lambda i: (0, 0)),
            pl.BlockSpec((HIDDEN, HIDDEN), lambda i: (0, 0)),
            pl.BlockSpec((8 * 128, HIDDEN), lambda i: (0, 0)),
            pl.BlockSpec((8 * HIDDEN, 128), lambda i: (0, 0)),
        ],
        out_specs=pl.BlockSpec((1, pk, 128), lambda i: (i, 0, 0)),
        out_shape=jax.ShapeDtypeStruct((grid, pk, 128), jnp.float32),
    )(ell, xp, w1n, w2n, rcat, ucat)


def _tc_combine(pa, pb):
    rows = pa.shape[1]
    blk = 128

    def body(a_ref, b_ref, o_ref):
        o_ref[...] = a_ref[0] + a_ref[1] + b_ref[0] + b_ref[1]

    return pl.pallas_call(
        body,
        grid=(rows // blk,),
        in_specs=[
            pl.BlockSpec((2, blk, 128), lambda i: (0, i, 0)),
            pl.BlockSpec((2, blk, 128), lambda i: (0, i, 0)),
        ],
        out_specs=pl.BlockSpec((blk, 128), lambda i: (i, 0)),
        out_shape=jax.ShapeDtypeStruct((rows, 128), jnp.float32),
    )(pa, pb)


def kernel(edge_src, edge_dst, node_features, edge_sh, edge_length_embedded,
           num_neighbors, W1, W2):
    E = edge_src.shape[0]
    w1n = (W1 * np.float32(1.0 / np.sqrt(3.0))).astype(jnp.float32)
    scale = np.float32(np.sqrt(2.0) / np.sqrt(float(HIDDEN)) / np.sqrt(float(D_IN)))
    w2n = W2 * (scale / jnp.sqrt(jnp.float32(num_neighbors)))

    lsh = jnp.concatenate(
        [edge_length_embedded, edge_sh], axis=1
    ).astype(jnp.bfloat16)
    w1p = jnp.pad(w1n, ((0, 1), (0, 0)))

    Eh = E // 2
    src = edge_src.astype(jnp.int32)
    dst = edge_dst.astype(jnp.int32)
    parts = []
    efps = []
    for hf in range(2):
        xp = _sc_gather(node_features, src, hf * Eh, Eh)
        efps.append(_tc_dense(lsh, xp, w1p, w2n, block_base=hf * (Eh // 2000)))
    for hf in range(2):
        parts.append(_sc_scatter_add(efps[hf], dst, hf * Eh, N_NODES))
    comb = _tc_combine(*parts)
    return comb.reshape(-1)[: N_NODES * D_OUT].reshape(N_NODES, D_OUT)
